# baseline (speedup 1.0000x reference)
import functools

import jax
import jax.numpy as jnp
from jax import lax
from jax.experimental import pallas as pl
from jax.experimental.pallas import tpu as pltpu
from jax.experimental.pallas import tpu_sc as plsc

_R = 16
_R3 = _R * _R * _R
_L = 16
_CHUNK = 2048
_K = 2


def _perm16(x, idx):
    dn = lax.GatherDimensionNumbers(
        offset_dims=(), collapsed_slice_dims=(0,), start_index_map=(0,))
    return lax.gather(x, idx[:, None], dn, (1,),
                      mode=lax.GatherScatterMode.PROMISE_IN_BOUNDS)


def _coords_body(coords_ref, nc_ref, pos_ref):
    c = coords_ref[0]
    mean = jnp.mean(c, axis=1, keepdims=True)
    cc = c - mean
    nrm = jnp.sqrt(jnp.sum(cc * cc, axis=0, keepdims=True))
    denom = jnp.max(nrm) * 2.0
    nc = cc / denom + 0.5
    nc = jnp.clip(nc * float(_R), 0.0, float(_R - 1))
    nc_ref[0] = nc
    v = jnp.round(nc).astype(jnp.int32)
    pos_ref[0] = (v[0:1] + v[1:2] * _R + v[2:3] * (_R * _R))


def _feat_t_body(x_ref, o_ref):
    o_ref[0, 0] = x_ref[0].T


def _sc_scatter_body(pos_hbm, feat_hbm, out_hbm, posbuf, featbuf, accbuf):
    B, G, NF = feat_hbm.shape
    N = NF // _L
    nchunks = N // _CHUNK
    wid = lax.axis_index("c") * 16 + lax.axis_index("s")
    neginf = jnp.full((_L,), -jnp.inf, jnp.float32)

    for t in range(8):
        opmin = t >= 4
        tb = (t % 4) * 2 + (wid >> 4)
        g = wid & 15

        def initbody(j, _):
            for r in range(16):
                accbuf[pl.ds((j * 16 + r) * _L, _L)] = neginf
            return 0
        lax.fori_loop(0, _R3 // _L, initbody, 0)

        pltpu.sync_copy(pos_hbm.at[tb], posbuf)

        for ck in range(nchunks):
            pltpu.sync_copy(
                feat_hbm.at[tb, g, pl.ds(ck * _CHUNK * _L, _CHUNK * _L)],
                featbuf)

            def blkbody(q, _):
                base = q * _L
                pv = posbuf[pl.ds(ck * _CHUNK + base, _L)]
                for sub in range(_L // _K):
                    ks = []
                    vs = []
                    for j in range(_K):
                        jj = sub * _K + j
                        k = pv[jj]
                        v = featbuf[pl.ds((base + jj) * _L, _L)]
                        if opmin:
                            v = -v
                        for i in range(j):
                            v = jnp.maximum(
                                v, jnp.where(ks[i] == k, vs[i], neginf))
                        ks.append(k)
                        vs.append(v)
                    news = [jnp.maximum(accbuf[pl.ds(ks[j] * _L, _L)], vs[j])
                            for j in range(_K)]
                    for j in range(_K):
                        accbuf[pl.ds(ks[j] * _L, _L)] = news[j]
                return 0

            lax.fori_loop(0, _CHUNK // _L, blkbody, 0)

        pltpu.sync_copy(
            accbuf, out_hbm.at[1 if opmin else 0, tb, g])


def _shift_left(x, k, fill):
    pad = jnp.full((x.shape[0], k), fill, x.dtype)
    return jnp.concatenate([x[:, k:], pad], axis=1)


def _suffix_mm_body(bmax_ref, bmin_ref, w_ref, b_ref, out_ref, s_ref, sq_ref):
    bm = bmax_ref[0]
    bn = bmin_ref[0]
    pres = bm[0:1, :] > -jnp.inf
    k = 1
    while k < _R3:
        bm = jnp.maximum(bm, _shift_left(bm, k, -jnp.inf))
        bn = jnp.maximum(bn, _shift_left(bn, k, -jnp.inf))
        k *= 2
    iota = lax.broadcasted_iota(jnp.int32, (1, _R3), 1)
    vmin = jnp.min(jnp.where(pres, iota, _R3))
    mask = pres & (iota != vmin)
    bm = jnp.where(mask, bm, 0.0)
    bn = jnp.where(mask, -bn, 0.0)
    fea = jnp.concatenate([bm, bn], axis=0)
    out = jax.lax.dot_general(w_ref[...], fea,
                              (((1,), (0,)), ((), ())),
                              preferred_element_type=jnp.float32)
    out = out + b_ref[:, 0:1]
    out_ref[0] = out
    c = out.shape[0]
    s_ref[0] = jnp.broadcast_to(jnp.sum(out, axis=1, keepdims=True), (c, 8))
    sq_ref[0] = jnp.broadcast_to(jnp.sum(out * out, axis=1, keepdims=True),
                                 (c, 8))


def _bn_swish_body(x_ref, scale_ref, shift_ref, y_ref):
    x = x_ref[0]
    y = x * scale_ref[:, 0:1] + shift_ref[:, 0:1]
    y_ref[0] = y * jax.nn.sigmoid(y)


def kernel(features, coords, W, b, gamma, beta):
    B, C, N = features.shape
    f32 = jnp.float32

    nc, pos = pl.pallas_call(
        _coords_body,
        grid=(B,),
        in_specs=[pl.BlockSpec((1, 3, N), lambda i: (i, 0, 0))],
        out_specs=[pl.BlockSpec((1, 3, N), lambda i: (i, 0, 0)),
                   pl.BlockSpec((1, 1, N), lambda i: (i, 0, 0))],
        out_shape=[jax.ShapeDtypeStruct((B, 3, N), f32),
                   jax.ShapeDtypeStruct((B, 1, N), jnp.int32)],
    )(coords)

    featT = pl.pallas_call(
        _feat_t_body,
        grid=(B, 16),
        in_specs=[pl.BlockSpec((1, 16, N), lambda i, j: (i, j, 0))],
        out_specs=pl.BlockSpec((1, 1, N, 16), lambda i, j: (i, j, 0, 0)),
        out_shape=jax.ShapeDtypeStruct((B, 16, N, 16), f32),
    )(features)

    mesh = plsc.VectorSubcoreMesh(core_axis_name="c", subcore_axis_name="s")
    sc_scatter = functools.partial(
        pl.kernel,
        mesh=mesh,
        out_type=jax.ShapeDtypeStruct((2, B, 16, _R3 * _L), f32),
        scratch_types=[
            pltpu.VMEM((N,), jnp.int32),
            pltpu.VMEM((_CHUNK * _L,), f32),
            pltpu.VMEM((_R3 * _L,), f32),
        ],
    )(_sc_scatter_body)
    bins = sc_scatter(pos.reshape(B, N), featT.reshape(B, 16, N * _L))
    binsT = bins.reshape(2, B, 16, _R3, _L).transpose(0, 1, 2, 4, 3)
    binsT = binsT.reshape(2, B, 16 * _L, _R3)

    bcol = jnp.broadcast_to(b.reshape(C, 1), (C, 8))

    outP, s, sq = pl.pallas_call(
        _suffix_mm_body,
        grid=(B,),
        in_specs=[pl.BlockSpec((1, C, _R3), lambda i: (i, 0, 0)),
                  pl.BlockSpec((1, C, _R3), lambda i: (i, 0, 0)),
                  pl.BlockSpec((C, 2 * C), lambda i: (0, 0)),
                  pl.BlockSpec((C, 8), lambda i: (0, 0))],
        out_specs=[pl.BlockSpec((1, C, _R3), lambda i: (i, 0, 0)),
                   pl.BlockSpec((1, C, 8), lambda i: (i, 0, 0)),
                   pl.BlockSpec((1, C, 8), lambda i: (i, 0, 0))],
        out_shape=[jax.ShapeDtypeStruct((B, C, _R3), f32),
                   jax.ShapeDtypeStruct((B, C, 8), f32),
                   jax.ShapeDtypeStruct((B, C, 8), f32)],
    )(binsT[0], binsT[1], W, bcol)

    cnt = float(B * _R3)
    mean = jnp.sum(s[:, :, 0], axis=0) / cnt
    var = jnp.sum(sq[:, :, 0], axis=0) / cnt - mean * mean
    scale = gamma / jnp.sqrt(var + 1e-5)
    shift = beta - mean * scale

    y = pl.pallas_call(
        _bn_swish_body,
        grid=(B,),
        in_specs=[pl.BlockSpec((1, C, _R3), lambda i: (i, 0, 0)),
                  pl.BlockSpec((C, 8), lambda i: (0, 0)),
                  pl.BlockSpec((C, 8), lambda i: (0, 0))],
        out_specs=pl.BlockSpec((1, C, _R3), lambda i: (i, 0, 0)),
        out_shape=jax.ShapeDtypeStruct((B, C, _R3), f32),
    )(outP, jnp.broadcast_to(scale.reshape(C, 1), (C, 8)),
      jnp.broadcast_to(shift.reshape(C, 1), (C, 8)))

    out = y.reshape(B, C, _R, _R, _R)
    return (out, nc.reshape(B, 3, N))


# speedup vs baseline: 1.0301x; 1.0301x over previous
import functools

import jax
import jax.numpy as jnp
from jax import lax
from jax.experimental import pallas as pl
from jax.experimental.pallas import tpu as pltpu
from jax.experimental.pallas import tpu_sc as plsc

_R = 16
_R3 = _R * _R * _R
_L = 16
_CHUNK = 2048
_K = 8


def _perm16(x, idx):
    dn = lax.GatherDimensionNumbers(
        offset_dims=(), collapsed_slice_dims=(0,), start_index_map=(0,))
    return lax.gather(x, idx[:, None], dn, (1,),
                      mode=lax.GatherScatterMode.PROMISE_IN_BOUNDS)


def _coords_body(coords_ref, nc_ref, pos_ref):
    c = coords_ref[0]
    mean = jnp.mean(c, axis=1, keepdims=True)
    cc = c - mean
    nrm = jnp.sqrt(jnp.sum(cc * cc, axis=0, keepdims=True))
    denom = jnp.max(nrm) * 2.0
    nc = cc / denom + 0.5
    nc = jnp.clip(nc * float(_R), 0.0, float(_R - 1))
    nc_ref[0] = nc
    v = jnp.round(nc).astype(jnp.int32)
    pos_ref[0] = (v[0:1] + v[1:2] * _R + v[2:3] * (_R * _R))


def _feat_t_body(x_ref, o_ref):
    o_ref[0, 0] = x_ref[0].T


def _sc_scatter_body(pos_hbm, feat_hbm, out_hbm, posbuf, featbuf, accbuf):
    B, G, NF = feat_hbm.shape
    N = NF // _L
    nchunks = N // _CHUNK
    wid = lax.axis_index("c") * 16 + lax.axis_index("s")
    neginf = jnp.full((_L,), -jnp.inf, jnp.float32)

    for t in range(8):
        opmin = t >= 4
        tb = (t % 4) * 2 + (wid >> 4)
        g = wid & 15

        def initbody(j, _):
            for r in range(16):
                accbuf[pl.ds((j * 16 + r) * _L, _L)] = neginf
            return 0
        lax.fori_loop(0, _R3 // _L, initbody, 0)

        pltpu.sync_copy(pos_hbm.at[tb], posbuf)

        for ck in range(nchunks):
            pltpu.sync_copy(
                feat_hbm.at[tb, g, pl.ds(ck * _CHUNK * _L, _CHUNK * _L)],
                featbuf)

            def blkbody(q, _):
                base = q * _L
                pv = posbuf[pl.ds(ck * _CHUNK + base, _L)]
                for sub in range(_L // _K):
                    ks = []
                    vs = []
                    for j in range(_K):
                        jj = sub * _K + j
                        k = pv[jj]
                        v = featbuf[pl.ds((base + jj) * _L, _L)]
                        if opmin:
                            v = -v
                        for i in range(j):
                            v = jnp.maximum(
                                v, jnp.where(ks[i] == k, vs[i], neginf))
                        ks.append(k)
                        vs.append(v)
                    news = [jnp.maximum(accbuf[pl.ds(ks[j] * _L, _L)], vs[j])
                            for j in range(_K)]
                    for j in range(_K):
                        accbuf[pl.ds(ks[j] * _L, _L)] = news[j]
                return 0

            lax.fori_loop(0, _CHUNK // _L, blkbody, 0)

        pltpu.sync_copy(
            accbuf, out_hbm.at[1 if opmin else 0, tb, g])


def _shift_left(x, k, fill):
    pad = jnp.full((x.shape[0], k), fill, x.dtype)
    return jnp.concatenate([x[:, k:], pad], axis=1)


def _suffix_mm_body(bmax_ref, bmin_ref, w_ref, b_ref, out_ref, s_ref, sq_ref):
    bm = bmax_ref[0]
    bn = bmin_ref[0]
    pres = bm[0:1, :] > -jnp.inf
    k = 1
    while k < _R3:
        bm = jnp.maximum(bm, _shift_left(bm, k, -jnp.inf))
        bn = jnp.maximum(bn, _shift_left(bn, k, -jnp.inf))
        k *= 2
    iota = lax.broadcasted_iota(jnp.int32, (1, _R3), 1)
    vmin = jnp.min(jnp.where(pres, iota, _R3))
    mask = pres & (iota != vmin)
    bm = jnp.where(mask, bm, 0.0)
    bn = jnp.where(mask, -bn, 0.0)
    fea = jnp.concatenate([bm, bn], axis=0)
    out = jax.lax.dot_general(w_ref[...], fea,
                              (((1,), (0,)), ((), ())),
                              preferred_element_type=jnp.float32)
    out = out + b_ref[:, 0:1]
    out_ref[0] = out
    c = out.shape[0]
    s_ref[0] = jnp.broadcast_to(jnp.sum(out, axis=1, keepdims=True), (c, 8))
    sq_ref[0] = jnp.broadcast_to(jnp.sum(out * out, axis=1, keepdims=True),
                                 (c, 8))


def _bn_swish_body(x_ref, scale_ref, shift_ref, y_ref):
    x = x_ref[0]
    y = x * scale_ref[:, 0:1] + shift_ref[:, 0:1]
    y_ref[0] = y * jax.nn.sigmoid(y)


def kernel(features, coords, W, b, gamma, beta):
    B, C, N = features.shape
    f32 = jnp.float32

    nc, pos = pl.pallas_call(
        _coords_body,
        grid=(B,),
        in_specs=[pl.BlockSpec((1, 3, N), lambda i: (i, 0, 0))],
        out_specs=[pl.BlockSpec((1, 3, N), lambda i: (i, 0, 0)),
                   pl.BlockSpec((1, 1, N), lambda i: (i, 0, 0))],
        out_shape=[jax.ShapeDtypeStruct((B, 3, N), f32),
                   jax.ShapeDtypeStruct((B, 1, N), jnp.int32)],
    )(coords)

    featT = pl.pallas_call(
        _feat_t_body,
        grid=(B, 16),
        in_specs=[pl.BlockSpec((1, 16, N), lambda i, j: (i, j, 0))],
        out_specs=pl.BlockSpec((1, 1, N, 16), lambda i, j: (i, j, 0, 0)),
        out_shape=jax.ShapeDtypeStruct((B, 16, N, 16), f32),
    )(features)

    mesh = plsc.VectorSubcoreMesh(core_axis_name="c", subcore_axis_name="s")
    sc_scatter = functools.partial(
        pl.kernel,
        mesh=mesh,
        out_type=jax.ShapeDtypeStruct((2, B, 16, _R3 * _L), f32),
        scratch_types=[
            pltpu.VMEM((N,), jnp.int32),
            pltpu.VMEM((_CHUNK * _L,), f32),
            pltpu.VMEM((_R3 * _L,), f32),
        ],
    )(_sc_scatter_body)
    bins = sc_scatter(pos.reshape(B, N), featT.reshape(B, 16, N * _L))
    binsT = bins.reshape(2, B, 16, _R3, _L).transpose(0, 1, 2, 4, 3)
    binsT = binsT.reshape(2, B, 16 * _L, _R3)

    bcol = jnp.broadcast_to(b.reshape(C, 1), (C, 8))

    outP, s, sq = pl.pallas_call(
        _suffix_mm_body,
        grid=(B,),
        in_specs=[pl.BlockSpec((1, C, _R3), lambda i: (i, 0, 0)),
                  pl.BlockSpec((1, C, _R3), lambda i: (i, 0, 0)),
                  pl.BlockSpec((C, 2 * C), lambda i: (0, 0)),
                  pl.BlockSpec((C, 8), lambda i: (0, 0))],
        out_specs=[pl.BlockSpec((1, C, _R3), lambda i: (i, 0, 0)),
                   pl.BlockSpec((1, C, 8), lambda i: (i, 0, 0)),
                   pl.BlockSpec((1, C, 8), lambda i: (i, 0, 0))],
        out_shape=[jax.ShapeDtypeStruct((B, C, _R3), f32),
                   jax.ShapeDtypeStruct((B, C, 8), f32),
                   jax.ShapeDtypeStruct((B, C, 8), f32)],
    )(binsT[0], binsT[1], W, bcol)

    cnt = float(B * _R3)
    mean = jnp.sum(s[:, :, 0], axis=0) / cnt
    var = jnp.sum(sq[:, :, 0], axis=0) / cnt - mean * mean
    scale = gamma / jnp.sqrt(var + 1e-5)
    shift = beta - mean * scale

    y = pl.pallas_call(
        _bn_swish_body,
        grid=(B,),
        in_specs=[pl.BlockSpec((1, C, _R3), lambda i: (i, 0, 0)),
                  pl.BlockSpec((C, 8), lambda i: (0, 0)),
                  pl.BlockSpec((C, 8), lambda i: (0, 0))],
        out_specs=pl.BlockSpec((1, C, _R3), lambda i: (i, 0, 0)),
        out_shape=jax.ShapeDtypeStruct((B, C, _R3), f32),
    )(outP, jnp.broadcast_to(scale.reshape(C, 1), (C, 8)),
      jnp.broadcast_to(shift.reshape(C, 1), (C, 8)))

    out = y.reshape(B, C, _R, _R, _R)
    return (out, nc.reshape(B, 3, N))
